# baseline (speedup 1.0000x reference)
import jax
import jax.numpy as jnp
from jax import lax
from jax.experimental import pallas as pl
from jax.experimental.pallas import tpu as pltpu
from jax.experimental.pallas import tpu_sc as plsc

OBJ = 1024
ATTR = 8
DIM = 256
BATCH = 4
SEQ = OBJ * ATTR

BATCH_SC = 2

_info = plsc.get_sparse_core_info()
_NC, _NS, _L = _info.num_cores, _info.num_subcores, _info.num_lanes
_NW = _NC * _NS
_ROWS_W = SEQ // _NW
_CHUNK = 128

_BLOCK_S = 512
_TROWS = _BLOCK_S // ATTR


def _sc_body(table_hbm, idx_hbm, out_hbm, idx_v, rows_v, sem, wsem):
    wid = lax.axis_index("s") * _NC + lax.axis_index("c")
    base = wid * _ROWS_W
    pltpu.sync_copy(idx_hbm.at[wid], idx_v)
    gathers = [
        pltpu.async_copy(
            table_hbm.at[idx_v.at[c]],
            rows_v.at[pl.ds(c * _CHUNK, _CHUNK)],
            sem,
        )
        for c in range(_ROWS_W // _CHUNK)
    ]
    for cp in gathers:
        cp.wait()
    writes = [
        pltpu.async_copy(rows_v, out_hbm.at[b, pl.ds(base, _ROWS_W)], wsem)
        for b in range(BATCH_SC)
    ]
    for w in writes:
        w.wait()


def _tc_body(table_ref, out_ref):
    t = table_ref[...]
    e = jnp.broadcast_to(t[:, None, :], (_TROWS, ATTR, DIM))
    out_ref[0] = e.reshape(_BLOCK_S, DIM)


def kernel(x, E_object_index):
    del x
    idx = (jnp.arange(SEQ, dtype=jnp.int32) // ATTR).reshape(
        _NW, _ROWS_W // _CHUNK, _CHUNK
    )
    sc_run = pl.kernel(
        _sc_body,
        out_type=jax.ShapeDtypeStruct((BATCH_SC, SEQ, DIM), jnp.float32),
        mesh=plsc.VectorSubcoreMesh(core_axis_name="c", subcore_axis_name="s"),
        scratch_types=[
            pltpu.VMEM((_ROWS_W // _CHUNK, _CHUNK), jnp.int32),
            pltpu.VMEM((_ROWS_W, DIM), jnp.float32),
            pltpu.SemaphoreType.DMA,
            pltpu.SemaphoreType.DMA,
        ],
    )
    sc_out = sc_run(E_object_index, idx)
    tc_out = pl.pallas_call(
        _tc_body,
        grid=(SEQ // _BLOCK_S, BATCH - BATCH_SC),
        in_specs=[pl.BlockSpec((_TROWS, DIM), lambda i, b: (i, 0))],
        out_specs=pl.BlockSpec((1, _BLOCK_S, DIM), lambda i, b: (b, i, 0)),
        out_shape=jax.ShapeDtypeStruct((BATCH - BATCH_SC, SEQ, DIM), jnp.float32),
    )(E_object_index)
    return jnp.concatenate([sc_out, tc_out], axis=0)
